# baseline (speedup 1.0000x reference)
import functools

import jax
import jax.numpy as jnp
from jax import lax
from jax.experimental import pallas as pl
from jax.experimental.pallas import tpu as pltpu
from jax.experimental.pallas import tpu_sc as plsc

N_ROWS = 100001
D = 128
B_ROWS = 1024
SEQ = 200
NUM_WORKERS = 32
ROWS_PER_W = B_ROWS // NUM_WORKERS
N_CHUNKS = ROWS_PER_W
NBUF = 4
SLACK = 2

_mesh = plsc.VectorSubcoreMesh(core_axis_name="c", subcore_axis_name="s")


@functools.partial(
    pl.kernel,
    mesh=_mesh,
    compiler_params=pltpu.CompilerParams(use_tc_tiling_on_sc=True),
    out_type=jax.ShapeDtypeStruct((B_ROWS, SEQ, D), jnp.float32),
    scratch_types=(
        [pltpu.VMEM((SEQ,), jnp.int32) for _ in range(NBUF)]
        + [pltpu.VMEM((SEQ, D), jnp.float32) for _ in range(NBUF)]
        + [pltpu.SemaphoreType.DMA for _ in range(3 * NBUF)]
    ),
)
def _gather_kernel(idx_hbm, table_hbm, out_hbm, *bufs):
    idxb = bufs[:NBUF]
    rows = bufs[NBUF:2 * NBUF]
    isem = bufs[2 * NBUF:3 * NBUF]
    gsem = bufs[3 * NBUF:4 * NBUF]
    ssem = bufs[4 * NBUF:]
    wid = lax.axis_index("s") * 2 + lax.axis_index("c")
    base = wid * ROWS_PER_W

    def stage_start(g, b):
        pltpu.async_copy(idx_hbm.at[base + g], idxb[b], isem[b])

    def stage_wait(g, b):
        pltpu.make_async_copy(idx_hbm.at[base + g], idxb[b], isem[b]).wait()

    def gather_start(g, b):
        pltpu.async_copy(table_hbm.at[idxb[b]], rows[b], gsem[b])

    def gather_wait(g, b):
        pltpu.make_async_copy(table_hbm.at[idxb[b]], rows[b], gsem[b]).wait()

    def store_start(g, b):
        pltpu.async_copy(rows[b], out_hbm.at[base + g], ssem[b])

    def store_wait(g, b):
        pltpu.make_async_copy(rows[b], out_hbm.at[base + g], ssem[b]).wait()

    for c in range(NBUF):
        stage_start(c, c)
    for c in range(NBUF):
        stage_wait(c, c)
        gather_start(c, c)

    def outer(go, carry):
        for j in range(NBUF):
            g = NBUF * go + j
            b = j
            b2 = (j - SLACK) % NBUF
            gather_wait(g, b)
            store_start(g, b)

            @pl.when(g + NBUF < N_CHUNKS)
            def _():
                stage_start(g + NBUF, b)

            @pl.when(g >= SLACK)
            def _():
                store_wait(g - SLACK, b2)

            @pl.when((g >= SLACK) & (g - SLACK + NBUF < N_CHUNKS))
            def _():
                stage_wait(g - SLACK + NBUF, b2)
                gather_start(g - SLACK + NBUF, b2)

        return carry

    lax.fori_loop(0, N_CHUNKS // NBUF, outer, 0)

    for g in range(N_CHUNKS - SLACK, N_CHUNKS):
        store_wait(g, g % NBUF)


def kernel(batch, emb_table):
    return _gather_kernel(batch, emb_table)


# speedup vs baseline: 1.0014x; 1.0014x over previous
import functools

import jax
import jax.numpy as jnp
from jax import lax
from jax.experimental import pallas as pl
from jax.experimental.pallas import tpu as pltpu
from jax.experimental.pallas import tpu_sc as plsc

N_ROWS = 100001
D = 128
B_ROWS = 1024
SEQ = 200
NUM_WORKERS = 32
ROWS_PER_W = B_ROWS // NUM_WORKERS
N_CHUNKS = ROWS_PER_W
NBUF = 4
SLACK = 2

_mesh = plsc.VectorSubcoreMesh(core_axis_name="c", subcore_axis_name="s")


@functools.partial(
    pl.kernel,
    mesh=_mesh,
    out_type=jax.ShapeDtypeStruct((B_ROWS, SEQ, D), jnp.float32),
    scratch_types=(
        [pltpu.VMEM((SEQ,), jnp.int32) for _ in range(NBUF)]
        + [pltpu.VMEM((SEQ, D), jnp.float32) for _ in range(NBUF)]
        + [pltpu.SemaphoreType.DMA for _ in range(3 * NBUF)]
    ),
)
def _gather_kernel(idx_hbm, table_hbm, out_hbm, *bufs):
    idxb = bufs[:NBUF]
    rows = bufs[NBUF:2 * NBUF]
    isem = bufs[2 * NBUF:3 * NBUF]
    gsem = bufs[3 * NBUF:4 * NBUF]
    ssem = bufs[4 * NBUF:]
    wid = lax.axis_index("s") * 2 + lax.axis_index("c")
    base = wid * ROWS_PER_W

    def stage_start(g, b):
        pltpu.async_copy(idx_hbm.at[base + g], idxb[b], isem[b])

    def stage_wait(g, b):
        pltpu.make_async_copy(idx_hbm.at[base + g], idxb[b], isem[b]).wait()

    def gather_start(g, b):
        pltpu.async_copy(table_hbm.at[idxb[b]], rows[b], gsem[b])

    def gather_wait(g, b):
        pltpu.make_async_copy(table_hbm.at[idxb[b]], rows[b], gsem[b]).wait()

    def store_start(g, b):
        pltpu.async_copy(rows[b], out_hbm.at[base + g], ssem[b])

    def store_wait(g, b):
        pltpu.make_async_copy(rows[b], out_hbm.at[base + g], ssem[b]).wait()

    for c in range(NBUF):
        stage_start(c, c)
    for c in range(NBUF):
        stage_wait(c, c)
        gather_start(c, c)

    def outer(go, carry):
        for j in range(NBUF):
            g = NBUF * go + j
            b = j
            b2 = (j - SLACK) % NBUF
            gather_wait(g, b)
            store_start(g, b)

            @pl.when(g + NBUF < N_CHUNKS)
            def _():
                stage_start(g + NBUF, b)

            @pl.when(g >= SLACK)
            def _():
                store_wait(g - SLACK, b2)

            @pl.when((g >= SLACK) & (g - SLACK + NBUF < N_CHUNKS))
            def _():
                stage_wait(g - SLACK + NBUF, b2)
                gather_start(g - SLACK + NBUF, b2)

        return carry

    lax.fori_loop(0, N_CHUNKS // NBUF, outer, 0)

    for g in range(N_CHUNKS - SLACK, N_CHUNKS):
        store_wait(g, g % NBUF)


def kernel(batch, emb_table):
    return _gather_kernel(batch, emb_table)


# speedup vs baseline: 1.0022x; 1.0008x over previous
import functools

import jax
import jax.numpy as jnp
from jax import lax
from jax.experimental import pallas as pl
from jax.experimental.pallas import tpu as pltpu
from jax.experimental.pallas import tpu_sc as plsc

N_ROWS = 100001
D = 128
B_ROWS = 1024
SEQ = 200
NUM_WORKERS = 32
ROWS_PER_W = B_ROWS // NUM_WORKERS
N_CHUNKS = ROWS_PER_W
NBUF = 4
SLACK = 2

_mesh = plsc.VectorSubcoreMesh(core_axis_name="c", subcore_axis_name="s")


@functools.partial(
    pl.kernel,
    mesh=_mesh,
    compiler_params=pltpu.CompilerParams(disable_bounds_checks=True),
    out_type=jax.ShapeDtypeStruct((B_ROWS, SEQ, D), jnp.float32),
    scratch_types=(
        [pltpu.VMEM((SEQ,), jnp.int32) for _ in range(NBUF)]
        + [pltpu.VMEM((SEQ, D), jnp.float32) for _ in range(NBUF)]
        + [pltpu.SemaphoreType.DMA for _ in range(3 * NBUF)]
    ),
)
def _gather_kernel(idx_hbm, table_hbm, out_hbm, *bufs):
    idxb = bufs[:NBUF]
    rows = bufs[NBUF:2 * NBUF]
    isem = bufs[2 * NBUF:3 * NBUF]
    gsem = bufs[3 * NBUF:4 * NBUF]
    ssem = bufs[4 * NBUF:]
    wid = lax.axis_index("s") * 2 + lax.axis_index("c")
    base = wid * ROWS_PER_W

    def stage_start(g, b):
        pltpu.async_copy(idx_hbm.at[base + g], idxb[b], isem[b])

    def stage_wait(g, b):
        pltpu.make_async_copy(idx_hbm.at[base + g], idxb[b], isem[b]).wait()

    def gather_start(g, b):
        pltpu.async_copy(table_hbm.at[idxb[b]], rows[b], gsem[b])

    def gather_wait(g, b):
        pltpu.make_async_copy(table_hbm.at[idxb[b]], rows[b], gsem[b]).wait()

    def store_start(g, b):
        pltpu.async_copy(rows[b], out_hbm.at[base + g], ssem[b])

    def store_wait(g, b):
        pltpu.make_async_copy(rows[b], out_hbm.at[base + g], ssem[b]).wait()

    for c in range(NBUF):
        stage_start(c, c)
    for c in range(NBUF):
        stage_wait(c, c)
        gather_start(c, c)

    def outer(go, carry):
        for j in range(NBUF):
            g = NBUF * go + j
            b = j
            b2 = (j - SLACK) % NBUF
            gather_wait(g, b)
            store_start(g, b)

            @pl.when(g + NBUF < N_CHUNKS)
            def _():
                stage_start(g + NBUF, b)

            @pl.when(g >= SLACK)
            def _():
                store_wait(g - SLACK, b2)

            @pl.when((g >= SLACK) & (g - SLACK + NBUF < N_CHUNKS))
            def _():
                stage_wait(g - SLACK + NBUF, b2)
                gather_start(g - SLACK + NBUF, b2)

        return carry

    lax.fori_loop(0, N_CHUNKS // NBUF, outer, 0)

    for g in range(N_CHUNKS - SLACK, N_CHUNKS):
        store_wait(g, g % NBUF)


def kernel(batch, emb_table):
    return _gather_kernel(batch, emb_table)
